# baseline (speedup 1.0000x reference)
import jax
import jax.numpy as jnp
from jax import lax
from jax.experimental import pallas as pl
from jax.experimental.pallas import tpu as pltpu
from jax.experimental.pallas import tpu_sc as plsc

B, H, W = 8, 512, 512
V = 64
NVOX = V * V * V
NROWS = NVOX // 2 + 8
PADROW = NVOX // 2
NC, NS = 2, 16
PTS_TILE = NVOX // NS
CHUNK = 2048
OCH = 512
NSUB = PTS_TILE // CHUNK
ZROWS = NROWS // NS
MAGIC = 12582912.0


def _body(rgbd, zer, out,
          dA, dB, rA, rB, gA, gB, blA, blB,
          srcA, srcB, idxA, idxB, axtab, acctA, acctB,
          oA0, oA1, oA2, oA3, oB0, oB1, oB2, oB3,
          acc, semz, seminA, seminB, semscA, semscB,
          semacA, semacB, semoA, semoB):
    cid = lax.axis_index("c")
    sid = lax.axis_index("s")
    lane = lax.iota(jnp.int32, 16)
    cols = [lane * 0 + c for c in range(8)]

    dbufs = (dA, dB)
    rbufs = (rA, rB)
    gbufs = (gA, gB)
    blbufs = (blA, blB)
    srcs = (srcA, srcB)
    idxs = (idxA, idxB)
    accts = (acctA, acctB)
    obufs = ((oA0, oA1, oA2, oA3), (oB0, oB1, oB2, oB3))
    semin = (seminA, seminB)
    semsc = (semscA, semscB)
    semac = (semacA, semacB)
    semo = (semoA, semoB)

    @plsc.parallel_loop(0, 32)
    def initax(i):
        u = i * 16 + lane
        axtab[pl.ds(i * 16, 16)] = (u - 256).astype(jnp.float32) * 0.00390625

    pt_tile = sid * PTS_TILE

    def fire_inputs(b, sub, pa):
        rt = sid * 4 + (sub >> 1)
        ct0 = (sub & 1) * 2
        s = semin[pa]
        return [
            pltpu.async_copy(rgbd.at[b, 3, rt, pl.ds(ct0, 2)], dbufs[pa], s),
            pltpu.async_copy(rgbd.at[b, 0, rt, pl.ds(ct0, 2)], rbufs[pa], s),
            pltpu.async_copy(rgbd.at[b, 1, rt, pl.ds(ct0, 2)], gbufs[pa], s),
            pltpu.async_copy(rgbd.at[b, 2, rt, pl.ds(ct0, 2)], blbufs[pa], s),
        ]

    def batch_body(bi, carry):
        b = cid * 4 + bi

        zcp = pltpu.async_copy(zer, acc.at[pl.ds(sid * ZROWS, ZROWS)], semz)

        in_descs = {0: fire_inputs(b, 0, 0)}
        sc_descs = {}
        for sub in range(NSUB):
            pa = sub & 1
            if sub < NSUB - 1:
                in_descs[1 - pa] = fire_inputs(b, sub + 1, 1 - pa)
            for dsc in in_descs.pop(pa):
                dsc.wait()
            if sub >= 2:
                sc_descs.pop(pa).wait()

            rt = sid * 4 + (sub >> 1)
            ct0 = (sub & 1) * 2
            db, rb, gb, blb = dbufs[pa], rbufs[pa], gbufs[pa], blbufs[pa]
            src, idx = srcs[pa], idxs[pa]

            @plsc.parallel_loop(0, CHUNK // 16, unroll=4)
            def cbody(i):
                sl = pl.ds(i * 16, 16)
                jb = i >> 6
                ir = (i >> 3) & 7
                ic = pl.ds((i & 7) * 16, 16)
                d = db[jb, ir, ic]
                ax = axtab[pl.ds((ct0 + jb) * 128 + (i & 7) * 16, 16)]
                ay = (rt * 8 + ir - 256).astype(jnp.float32) * 0.00390625
                pnx = (ax * d + 2.0) * 15.75
                pny = (ay * d + 2.0) * 15.75
                pnz = (d + 2.0) * 15.75
                rnx = (pnx + MAGIC) - MAGIC
                rny = (pny + MAGIC) - MAGIC
                rnz = (pnz + MAGIC) - MAGIC
                inr = ((d > 0.0) & (d < 10.0)
                       & (rnx >= 0.0) & (rnx <= 63.0)
                       & (rny >= 0.0) & (rny <= 63.0)
                       & (rnz >= 0.0) & (rnz <= 63.0))
                flatf = rnx * 4096.0 + rny * 64.0 + rnz
                flatf = jnp.minimum(jnp.maximum(flatf, 0.0), 262143.0)
                flat = flatf.astype(jnp.int32)
                idx[sl] = jnp.where(inr, flat >> 1, PADROW)
                m1 = (flat & 1).astype(jnp.float32)
                m0 = 1.0 - m1
                rows = i * 16 + lane
                rv = rb[jb, ir, ic]
                gv = gb[jb, ir, ic]
                bv = blb[jb, ir, ic]
                plsc.store_scatter(src, [rows, cols[0]], m0)
                plsc.store_scatter(src, [rows, cols[1]], rv * m0)
                plsc.store_scatter(src, [rows, cols[2]], gv * m0)
                plsc.store_scatter(src, [rows, cols[3]], bv * m0)
                plsc.store_scatter(src, [rows, cols[4]], m1)
                plsc.store_scatter(src, [rows, cols[5]], rv * m1)
                plsc.store_scatter(src, [rows, cols[6]], gv * m1)
                plsc.store_scatter(src, [rows, cols[7]], bv * m1)

            if sub == 0:
                zcp.wait()
                plsc.subcore_barrier()
            sc_descs[pa] = pltpu.async_copy(
                src, acc.at[idx], semsc[pa], add=True)

        sc_descs.pop(0).wait()
        sc_descs.pop(1).wait()
        plsc.subcore_barrier()

        NOSUB = PTS_TILE // OCH
        ac_descs = {0: pltpu.async_copy(
            acc.at[pl.ds(sid * PTS_TILE // 2, OCH // 2)], accts[0], semac[0])}
        o_descs = {}
        for sub in range(NOSUB):
            pa = sub & 1
            vbase = sid * PTS_TILE + sub * OCH
            if sub < NOSUB - 1:
                nvb = vbase + OCH
                ac_descs[1 - pa] = pltpu.async_copy(
                    acc.at[pl.ds(nvb // 2, OCH // 2)], accts[1 - pa],
                    semac[1 - pa])
            ac_descs.pop(pa).wait()
            if sub >= 2:
                for dsc in o_descs.pop(pa):
                    dsc.wait()

            acct = accts[pa]
            o0, o1, o2, o3 = obufs[pa]

            @plsc.parallel_loop(0, OCH // 16, unroll=4)
            def obody(i):
                l = i * 16 + lane
                arow = l >> 1
                cb = (l & 1) * 4
                cnt = plsc.load_gather(acct, [arow, cb])
                rr = plsc.load_gather(acct, [arow, cb + 1])
                gg = plsc.load_gather(acct, [arow, cb + 2])
                bb = plsc.load_gather(acct, [arow, cb + 3])
                pos = cnt > 0.0
                occ = jnp.where(pos, 1.0, 0.0)
                rcp = occ / jnp.where(pos, cnt, 1.0)
                orow = i >> 2
                ocol = pl.ds((i & 3) * 16, 16)
                o0[orow, ocol] = occ
                o1[orow, ocol] = rr * rcp
                o2[orow, ocol] = gg * rcp
                o3[orow, ocol] = bb * rcp

            xs = vbase >> 12
            ys = (vbase >> 6) & 63
            o_descs[pa] = [
                pltpu.async_copy(o0, out.at[b, 0, xs, pl.ds(ys, 8), :], semo[pa]),
                pltpu.async_copy(o1, out.at[b, 1, xs, pl.ds(ys, 8), :], semo[pa]),
                pltpu.async_copy(o2, out.at[b, 2, xs, pl.ds(ys, 8), :], semo[pa]),
                pltpu.async_copy(o3, out.at[b, 3, xs, pl.ds(ys, 8), :], semo[pa]),
            ]

        for pa in (0, 1):
            for dsc in o_descs.pop(pa):
                dsc.wait()
        plsc.subcore_barrier()
        return carry

    lax.fori_loop(0, 4, batch_body, 0)


@jax.jit
def _voxelize_sc(rgbd_flat, zer):
    mesh = plsc.VectorSubcoreMesh(core_axis_name="c", subcore_axis_name="s")
    fbuf = lambda *s: pltpu.VMEM(s, jnp.float32)
    f = pl.kernel(
        _body,
        out_type=jax.ShapeDtypeStruct((B, 4, V, V, V), jnp.float32),
        mesh=mesh,
        compiler_params=pltpu.CompilerParams(
            needs_layout_passes=False, use_tc_tiling_on_sc=False),
        scratch_types=[
            fbuf(2, 8, 128), fbuf(2, 8, 128),
            fbuf(2, 8, 128), fbuf(2, 8, 128),
            fbuf(2, 8, 128), fbuf(2, 8, 128),
            fbuf(2, 8, 128), fbuf(2, 8, 128),
            fbuf(CHUNK, 8), fbuf(CHUNK, 8),
            pltpu.VMEM((CHUNK,), jnp.int32),
            pltpu.VMEM((CHUNK,), jnp.int32),
            fbuf(512),
            fbuf(OCH // 2, 8), fbuf(OCH // 2, 8),
            fbuf(8, 64), fbuf(8, 64), fbuf(8, 64), fbuf(8, 64),
            fbuf(8, 64), fbuf(8, 64), fbuf(8, 64), fbuf(8, 64),
            pltpu.VMEM_SHARED((NROWS, 8), jnp.float32),
        ] + [pltpu.SemaphoreType.DMA] * 9,
    )
    return f(rgbd_flat, zer)


def kernel(rgbd_images):
    xt = rgbd_images.reshape(B, 4, 64, 8, 4, 128).transpose(0, 1, 2, 4, 3, 5)
    zer = jnp.zeros((ZROWS, 8), jnp.float32)
    return _voxelize_sc(xt, zer)


# speedup vs baseline: 1.0047x; 1.0047x over previous
import jax
import jax.numpy as jnp
from jax import lax
from jax.experimental import pallas as pl
from jax.experimental.pallas import tpu as pltpu
from jax.experimental.pallas import tpu_sc as plsc

B, H, W = 8, 512, 512
V = 64
NVOX = V * V * V
NROWS = NVOX // 2 + 8
PADROW = NVOX // 2
NC, NS = 2, 16
PTS_TILE = NVOX // NS
CHUNK = 1024
OCH = 1024
NSUB = PTS_TILE // CHUNK
ZROWS = NROWS // NS
MAGIC = 12582912.0


def _body(rgbd, zer, out,
          dA, dB, rA, rB, gA, gB, blA, blB,
          srcA, srcB, idxA, idxB, axtab, acctA, acctB,
          oA0, oA1, oA2, oA3, oB0, oB1, oB2, oB3,
          acc, semz, seminA, seminB, semscA, semscB,
          semacA, semacB, semoA, semoB):
    cid = lax.axis_index("c")
    sid = lax.axis_index("s")
    lane = lax.iota(jnp.int32, 16)
    cols = [lane * 0 + c for c in range(8)]

    dbufs = (dA, dB)
    rbufs = (rA, rB)
    gbufs = (gA, gB)
    blbufs = (blA, blB)
    srcs = (srcA, srcB)
    idxs = (idxA, idxB)
    accts = (acctA, acctB)
    obufs = ((oA0, oA1, oA2, oA3), (oB0, oB1, oB2, oB3))
    semin = (seminA, seminB)
    semsc = (semscA, semscB)
    semac = (semacA, semacB)
    semo = (semoA, semoB)

    @plsc.parallel_loop(0, 32)
    def initax(i):
        u = i * 16 + lane
        axtab[pl.ds(i * 16, 16)] = (u - 256).astype(jnp.float32) * 0.00390625

    pt_tile = sid * PTS_TILE

    def fire_inputs(b, sub, pa):
        rt = sid * 4 + (sub >> 2)
        ct = sub & 3
        s = semin[pa]
        return [
            pltpu.async_copy(rgbd.at[b, 3, rt, ct], dbufs[pa], s),
            pltpu.async_copy(rgbd.at[b, 0, rt, ct], rbufs[pa], s),
            pltpu.async_copy(rgbd.at[b, 1, rt, ct], gbufs[pa], s),
            pltpu.async_copy(rgbd.at[b, 2, rt, ct], blbufs[pa], s),
        ]

    def batch_body(bi, carry):
        b = cid * 4 + bi

        zcp = pltpu.async_copy(zer, acc.at[pl.ds(sid * ZROWS, ZROWS)], semz)

        in_descs = {0: fire_inputs(b, 0, 0)}
        sc_descs = {}
        for sub in range(NSUB):
            pa = sub & 1
            if sub < NSUB - 1:
                in_descs[1 - pa] = fire_inputs(b, sub + 1, 1 - pa)
            for dsc in in_descs.pop(pa):
                dsc.wait()
            if sub >= 2:
                sc_descs.pop(pa).wait()

            rt = sid * 4 + (sub >> 2)
            ct = sub & 3
            db, rb, gb, blb = dbufs[pa], rbufs[pa], gbufs[pa], blbufs[pa]
            src, idx = srcs[pa], idxs[pa]

            @plsc.parallel_loop(0, CHUNK // 16, unroll=4)
            def cbody(i):
                sl = pl.ds(i * 16, 16)
                ir = i >> 3
                ic = pl.ds((i & 7) * 16, 16)
                d = db[ir, ic]
                ax = axtab[pl.ds(ct * 128 + (i & 7) * 16, 16)]
                ay = (rt * 8 + ir - 256).astype(jnp.float32) * 0.00390625
                pnx = (ax * d + 2.0) * 15.75
                pny = (ay * d + 2.0) * 15.75
                pnz = (d + 2.0) * 15.75
                rnx = (pnx + MAGIC) - MAGIC
                rny = (pny + MAGIC) - MAGIC
                rnz = (pnz + MAGIC) - MAGIC
                inr = ((d > 0.0) & (d < 10.0)
                       & (rnx >= 0.0) & (rnx <= 63.0)
                       & (rny >= 0.0) & (rny <= 63.0)
                       & (rnz >= 0.0) & (rnz <= 63.0))
                flatf = rnx * 4096.0 + rny * 64.0 + rnz
                flatf = jnp.minimum(jnp.maximum(flatf, 0.0), 262143.0)
                flat = flatf.astype(jnp.int32)
                idx[sl] = jnp.where(inr, flat >> 1, PADROW)
                m1 = (flat & 1).astype(jnp.float32)
                m0 = 1.0 - m1
                rows = i * 16 + lane
                rv = rb[ir, ic]
                gv = gb[ir, ic]
                bv = blb[ir, ic]
                plsc.store_scatter(src, [rows, cols[0]], m0)
                plsc.store_scatter(src, [rows, cols[1]], rv * m0)
                plsc.store_scatter(src, [rows, cols[2]], gv * m0)
                plsc.store_scatter(src, [rows, cols[3]], bv * m0)
                plsc.store_scatter(src, [rows, cols[4]], m1)
                plsc.store_scatter(src, [rows, cols[5]], rv * m1)
                plsc.store_scatter(src, [rows, cols[6]], gv * m1)
                plsc.store_scatter(src, [rows, cols[7]], bv * m1)

            if sub == 0:
                zcp.wait()
                plsc.subcore_barrier()
            sc_descs[pa] = pltpu.async_copy(
                src, acc.at[idx], semsc[pa], add=True)

        sc_descs.pop(0).wait()
        sc_descs.pop(1).wait()
        plsc.subcore_barrier()

        NOSUB = PTS_TILE // OCH
        ac_descs = {0: pltpu.async_copy(
            acc.at[pl.ds(sid * PTS_TILE // 2, OCH // 2)], accts[0], semac[0])}
        o_descs = {}
        for sub in range(NOSUB):
            pa = sub & 1
            vbase = sid * PTS_TILE + sub * OCH
            if sub < NOSUB - 1:
                nvb = vbase + OCH
                ac_descs[1 - pa] = pltpu.async_copy(
                    acc.at[pl.ds(nvb // 2, OCH // 2)], accts[1 - pa],
                    semac[1 - pa])
            ac_descs.pop(pa).wait()
            if sub >= 2:
                for dsc in o_descs.pop(pa):
                    dsc.wait()

            acct = accts[pa]
            o0, o1, o2, o3 = obufs[pa]

            @plsc.parallel_loop(0, OCH // 16, unroll=4)
            def obody(i):
                l = i * 16 + lane
                arow = l >> 1
                cb = (l & 1) * 4
                cnt = plsc.load_gather(acct, [arow, cb])
                rr = plsc.load_gather(acct, [arow, cb + 1])
                gg = plsc.load_gather(acct, [arow, cb + 2])
                bb = plsc.load_gather(acct, [arow, cb + 3])
                pos = cnt > 0.0
                occ = jnp.where(pos, 1.0, 0.0)
                rcp = occ / jnp.where(pos, cnt, 1.0)
                orow = i >> 2
                ocol = pl.ds((i & 3) * 16, 16)
                o0[orow, ocol] = occ
                o1[orow, ocol] = rr * rcp
                o2[orow, ocol] = gg * rcp
                o3[orow, ocol] = bb * rcp

            xs = vbase >> 12
            ys = (vbase >> 6) & 63
            o_descs[pa] = [
                pltpu.async_copy(o0, out.at[b, 0, xs, pl.ds(ys, 16), :], semo[pa]),
                pltpu.async_copy(o1, out.at[b, 1, xs, pl.ds(ys, 16), :], semo[pa]),
                pltpu.async_copy(o2, out.at[b, 2, xs, pl.ds(ys, 16), :], semo[pa]),
                pltpu.async_copy(o3, out.at[b, 3, xs, pl.ds(ys, 16), :], semo[pa]),
            ]

        for pa in (0, 1):
            for dsc in o_descs.pop(pa):
                dsc.wait()
        plsc.subcore_barrier()
        return carry

    lax.fori_loop(0, 4, batch_body, 0)


@jax.jit
def _voxelize_sc(rgbd_flat, zer):
    mesh = plsc.VectorSubcoreMesh(core_axis_name="c", subcore_axis_name="s")
    fbuf = lambda *s: pltpu.VMEM(s, jnp.float32)
    f = pl.kernel(
        _body,
        out_type=jax.ShapeDtypeStruct((B, 4, V, V, V), jnp.float32),
        mesh=mesh,
        compiler_params=pltpu.CompilerParams(
            needs_layout_passes=False, use_tc_tiling_on_sc=False),
        scratch_types=[
            fbuf(8, 128), fbuf(8, 128),
            fbuf(8, 128), fbuf(8, 128),
            fbuf(8, 128), fbuf(8, 128),
            fbuf(8, 128), fbuf(8, 128),
            fbuf(CHUNK, 8), fbuf(CHUNK, 8),
            pltpu.VMEM((CHUNK,), jnp.int32),
            pltpu.VMEM((CHUNK,), jnp.int32),
            fbuf(512),
            fbuf(OCH // 2, 8), fbuf(OCH // 2, 8),
            fbuf(16, 64), fbuf(16, 64), fbuf(16, 64), fbuf(16, 64),
            fbuf(16, 64), fbuf(16, 64), fbuf(16, 64), fbuf(16, 64),
            pltpu.VMEM_SHARED((NROWS, 8), jnp.float32),
        ] + [pltpu.SemaphoreType.DMA] * 9,
    )
    return f(rgbd_flat, zer)


def kernel(rgbd_images):
    xt = rgbd_images.reshape(B, 4, 64, 8, 4, 128).transpose(0, 1, 2, 4, 3, 5)
    zer = jnp.zeros((ZROWS, 8), jnp.float32)
    return _voxelize_sc(xt, zer)
